# baseline (speedup 1.0000x reference)
import functools
import jax
import jax.numpy as jnp
from jax import lax
from jax.experimental import pallas as pl
from jax.experimental.pallas import tpu as pltpu
from jax.experimental.pallas import tpu_sc as plsc

_B, _L, _D = 4, 8192, 1024
_TOTAL = _B * _L * _D
_POS_TOTAL = _L * _D
_NW = 32
_PER_W = _TOTAL // _NW
_CHUNK = 16 * 1024
_NCHUNK = _PER_W // _CHUNK


def _sc_body(x_hbm, pos_hbm, out_hbm, xbuf, pbuf):
    c = lax.axis_index("c")
    s = lax.axis_index("s")
    wid = s * 2 + c
    base = wid * _PER_W
    pbase = lax.rem(base, _POS_TOTAL)

    def chunk_body(i, carry):
        off = base + i * _CHUNK
        poff = pbase + i * _CHUNK
        pltpu.sync_copy(x_hbm.at[pl.ds(off, _CHUNK)], xbuf)
        pltpu.sync_copy(pos_hbm.at[pl.ds(poff, _CHUNK)], pbuf)

        @plsc.parallel_loop(0, _CHUNK // 16, 1, unroll=8)
        def add_body(k):
            sl = pl.ds(k * 16, 16)
            xbuf[sl] = xbuf[sl] + pbuf[sl]

        pltpu.sync_copy(xbuf, out_hbm.at[pl.ds(off, _CHUNK)])
        return carry

    lax.fori_loop(0, _NCHUNK, chunk_body, 0)


def kernel(x, pos_table):
    mesh = plsc.VectorSubcoreMesh(core_axis_name="c", subcore_axis_name="s")
    run = pl.kernel(
        _sc_body,
        mesh=mesh,
        out_type=jax.ShapeDtypeStruct((_TOTAL,), jnp.float32),
        scratch_types=[
            pltpu.VMEM((_CHUNK,), jnp.float32),
            pltpu.VMEM((_CHUNK,), jnp.float32),
        ],
    )
    out = run(x.reshape(-1), pos_table.reshape(-1))
    return out.reshape(x.shape)


# speedup vs baseline: 1.3636x; 1.3636x over previous
import jax
import jax.numpy as jnp
from jax import lax
from jax.experimental import pallas as pl
from jax.experimental.pallas import tpu as pltpu
from jax.experimental.pallas import tpu_sc as plsc

_B, _L, _D = 4, 8192, 1024
_TOTAL = _B * _L * _D
_POS_TOTAL = _L * _D
_NW = 32
_PER_W = _TOTAL // _NW
_CHUNK = 8192
_NCHUNK = _PER_W // _CHUNK
_NBUF = 4
_VREGS = _CHUNK // 16


def _sc_body(x_hbm, pos_hbm, out_hbm, *scratch):
    xb = scratch[0:_NBUF]
    pb = scratch[_NBUF:2 * _NBUF]
    inx = scratch[2 * _NBUF:3 * _NBUF]
    inp = scratch[3 * _NBUF:4 * _NBUF]
    outs = scratch[4 * _NBUF:5 * _NBUF]

    c = lax.axis_index("c")
    s = lax.axis_index("s")
    wid = s * 2 + c
    base = wid * _PER_W
    pbase = lax.rem(base, _POS_TOTAL)

    def in_copy(i, b):
        pltpu.async_copy(x_hbm.at[pl.ds(base + i * _CHUNK, _CHUNK)], xb[b], inx[b])
        pltpu.async_copy(pos_hbm.at[pl.ds(pbase + i * _CHUNK, _CHUNK)], pb[b], inp[b])

    def in_wait(i, b):
        pltpu.make_async_copy(x_hbm.at[pl.ds(base + i * _CHUNK, _CHUNK)], xb[b], inx[b]).wait()
        pltpu.make_async_copy(pos_hbm.at[pl.ds(pbase + i * _CHUNK, _CHUNK)], pb[b], inp[b]).wait()

    def out_copy(i, b):
        pltpu.async_copy(xb[b], out_hbm.at[pl.ds(base + i * _CHUNK, _CHUNK)], outs[b])

    def out_wait(i, b):
        pltpu.make_async_copy(xb[b], out_hbm.at[pl.ds(base + i * _CHUNK, _CHUNK)], outs[b]).wait()

    def compute(b):
        xref, pref = xb[b], pb[b]

        @plsc.parallel_loop(0, _VREGS, 1, unroll=16)
        def add_body(k):
            sl = pl.ds(k * 16, 16)
            plsc.addupdate(xref.at[sl], pref[sl])

    in_copy(0, 0)
    in_copy(1, 1)

    for b in range(_NBUF):
        i = b
        jp = (b + 2) % _NBUF
        if b < 2:
            in_copy(i + 2, jp)
        else:
            out_wait(i - 2, jp)
            in_copy(i + 2, jp)
        in_wait(i, b)
        compute(b)
        out_copy(i, b)

    def steady(g, carry):
        i0 = g * _NBUF
        for b in range(_NBUF):
            i = i0 + b
            jp = (b + 2) % _NBUF
            out_wait(i - 2, jp)
            in_copy(i + 2, jp)
            in_wait(i, b)
            compute(b)
            out_copy(i, b)
        return carry

    lax.fori_loop(1, _NCHUNK // _NBUF - 1, steady, 0)

    i0 = _NCHUNK - _NBUF
    for b in range(_NBUF):
        i = i0 + b
        jp = (b + 2) % _NBUF
        if b < 2:
            out_wait(i - 2, jp)
            in_copy(i + 2, jp)
        in_wait(i, b)
        compute(b)
        out_copy(i, b)

    for b in range(_NBUF):
        out_wait(i0 + b, b)


def kernel(x, pos_table):
    mesh = plsc.VectorSubcoreMesh(core_axis_name="c", subcore_axis_name="s")
    scratch = (
        [pltpu.VMEM((_CHUNK,), jnp.float32)] * (2 * _NBUF)
        + [pltpu.SemaphoreType.DMA] * (3 * _NBUF)
    )
    run = pl.kernel(
        _sc_body,
        mesh=mesh,
        out_type=jax.ShapeDtypeStruct((_TOTAL,), jnp.float32),
        scratch_types=scratch,
    )
    out = run(x.reshape(-1), pos_table.reshape(-1))
    return out.reshape(x.shape)


# speedup vs baseline: 6.1478x; 4.5085x over previous
import jax
import jax.numpy as jnp
from jax.experimental import pallas as pl


def _add_body(x_ref, pos_ref, out_ref):
    out_ref[...] = x_ref[...] + pos_ref[...]


def kernel(x, pos_table):
    B, L, D = x.shape
    BL = 2048
    num_l = L // BL
    grid = (num_l, B)
    return pl.pallas_call(
        _add_body,
        grid=grid,
        in_specs=[
            pl.BlockSpec((1, BL, D), lambda l, b: (b, l, 0)),
            pl.BlockSpec((BL, D), lambda l, b: (l, 0)),
        ],
        out_specs=pl.BlockSpec((1, BL, D), lambda l, b: (b, l, 0)),
        out_shape=jax.ShapeDtypeStruct((B, L, D), x.dtype),
    )(x, pos_table)
